# Initial kernel scaffold; baseline (speedup 1.0000x reference)
import jax
import jax.numpy as jnp
from jax.experimental import pallas as pl


def kernel(nodes, T, taus, B):
    raise NotImplementedError("write your pallas kernel here")


# speedup vs baseline: 1.8887x; 1.8887x over previous
import jax
import jax.numpy as jnp
from jax.experimental import pallas as pl
from jax.experimental.pallas import tpu as pltpu

RADIUS = 0.25


def _edge_kernel(lo_ref, hi_ref, pos_r_ref, pos_c_ref, out_ref, *, n, bj, b_count):
    b = pl.program_id(0)
    jb = pl.program_id(1)
    lo = lo_ref[b]
    hi = hi_ref[b]
    mx = hi_ref[0]
    for k in range(1, b_count):
        mx = jnp.maximum(mx, hi_ref[k])
    j0 = jb * bj
    active = (hi > j0) & (lo < j0 + bj) & (mx > 1)

    @pl.when(jnp.logical_not(active))
    def _():
        out_ref[...] = jnp.zeros((1, n, bj), jnp.float32)

    @pl.when(active)
    def _():
        pr = pos_r_ref[0]
        pc = pos_c_ref[0]
        acc = jnp.zeros((n, bj), jnp.float32)
        for k in range(3):
            d = pr[:, k : k + 1] - pc[k : k + 1, :]
            acc = acc + d * d
        w = (jnp.sqrt(acc) < RADIUS).astype(jnp.float32)
        row = jax.lax.broadcasted_iota(jnp.int32, (n, bj), 0)
        col = jax.lax.broadcasted_iota(jnp.int32, (n, bj), 1) + j0
        mask = (col >= lo) & (col < hi) & (row <= col)
        out_ref[0] = jnp.where(mask, w, 0.0)


def kernel(nodes, T, taus, B):
    B_s, N, _ = nodes.shape
    BJ = 512
    pos = nodes[:, :, 0:3]
    pos_c = jnp.transpose(pos, (0, 2, 1))
    lo = T.astype(jnp.int32)
    hi = (T + taus).astype(jnp.int32)

    import functools

    grid = (B_s, N // BJ)
    out = pl.pallas_call(
        functools.partial(_edge_kernel, n=N, bj=BJ, b_count=B_s),
        grid_spec=pltpu.PrefetchScalarGridSpec(
            num_scalar_prefetch=2,
            grid=grid,
            in_specs=[
                pl.BlockSpec((1, N, 3), lambda b, j, lo_r, hi_r: (b, 0, 0)),
                pl.BlockSpec((1, 3, BJ), lambda b, j, lo_r, hi_r: (b, 0, j)),
            ],
            out_specs=pl.BlockSpec((1, N, BJ), lambda b, j, lo_r, hi_r: (b, 0, j)),
        ),
        out_shape=jax.ShapeDtypeStruct((B_s, N, N), jnp.float32),
    )(lo, hi, pos, pos_c)
    return out
